# baseline (speedup 1.0000x reference)
import jax
import jax.numpy as jnp
from jax.experimental import pallas as pl

N = 10000
E = 320000
HIDDEN = 64
NUM_REGIONS = 8


def _node_transform(h, W, b):
    Np, F = h.shape
    H = W.shape[1]

    def body(h_ref, w_ref, b_ref, d_ref, bm_ref):
        hh = h_ref[...]
        wtop = w_ref[:F, :]
        wbot = w_ref[F:, :]
        bm = jnp.dot(hh, wbot, preferred_element_type=jnp.float32)
        d_ref[...] = (
            jnp.dot(hh, wtop, preferred_element_type=jnp.float32)
            - bm
            + b_ref[...]
        )
        bm_ref[...] = bm

    return pl.pallas_call(
        body,
        out_shape=(
            jax.ShapeDtypeStruct((Np, H), jnp.float32),
            jax.ShapeDtypeStruct((Np, H), jnp.float32),
        ),
    )(h, W, b.reshape(1, H))


def _edge_mlp(G1, G2, W2, b2):
    Ee, H = G1.shape
    BLK = 3200
    grid = Ee // BLK

    def body(g1_ref, g2_ref, w_ref, b_ref, o_ref):
        g = jnp.maximum(g1_ref[...] + g2_ref[...], 0.0)
        o_ref[...] = (
            jnp.dot(g, w_ref[...], preferred_element_type=jnp.float32)
            + b_ref[...]
        )

    return pl.pallas_call(
        body,
        grid=(grid,),
        in_specs=[
            pl.BlockSpec((BLK, H), lambda i: (i, 0)),
            pl.BlockSpec((BLK, H), lambda i: (i, 0)),
            pl.BlockSpec((H, H), lambda i: (0, 0)),
            pl.BlockSpec((1, H), lambda i: (0, 0)),
        ],
        out_specs=pl.BlockSpec((BLK, H), lambda i: (i, 0)),
        out_shape=jax.ShapeDtypeStruct((Ee, H), jnp.float32),
    )(G1, G2, W2, b2.reshape(1, H))


def _edge_conv_big(h, src, dst, W1, b1, W2, b2):
    D, B = _node_transform(h, W1, b1)
    G1 = jnp.take(D, dst, axis=0)
    G2 = jnp.take(B, src, axis=0)
    R = _edge_mlp(G1, G2, W2, b2)
    return jnp.maximum(jax.ops.segment_max(R, dst, num_segments=h.shape[0]), 0.0)


def _edge_conv_small(x, edge_index, W1, b1, W2, b2):
    src = edge_index[0]
    dst = edge_index[1]
    x_i = jnp.take(x, dst, axis=0)
    x_j = jnp.take(x, src, axis=0)
    m = jnp.concatenate([x_i, x_j - x_i], axis=1)
    m = jax.nn.relu(m @ W1 + b1) @ W2 + b2
    out = jax.ops.segment_max(m, dst, num_segments=x.shape[0])
    return jnp.maximum(out, 0.0)


def _kmeans_labels(pos, k, iters=10):
    cent = pos[:k]
    for _ in range(iters):
        d = jnp.sum((pos[:, None, :] - cent[None, :, :]) ** 2, axis=-1)
        labels = jnp.argmin(d, axis=1)
        sums = jax.ops.segment_sum(pos, labels, num_segments=k)
        counts = jax.ops.segment_sum(jnp.ones((pos.shape[0],), jnp.float32), labels, num_segments=k)
        cent = sums / jnp.maximum(counts, 1.0)[:, None]
    d = jnp.sum((pos[:, None, :] - cent[None, :, :]) ** 2, axis=-1)
    return jnp.argmin(d, axis=1)


def kernel(x, pos, edge_index, c1_W1, c1_b1, c1_W2, c1_b2, c2_W1, c2_b1, c2_W2, c2_b2, c3_W1, c3_b1, c3_W2, c3_b2, c4_W1, c4_b1, c4_W2, c4_b2, lin_W, lin_b):
    src = edge_index[0]
    dst = edge_index[1]
    h = jnp.concatenate([x, pos], axis=1)
    h = _edge_conv_big(h, src, dst, c1_W1, c1_b1, c1_W2, c1_b2)
    h = _edge_conv_big(h, src, dst, c2_W1, c2_b1, c2_W2, c2_b2)

    labels = _kmeans_labels(pos, NUM_REGIONS)
    counts = jax.ops.segment_sum(jnp.ones((h.shape[0],), jnp.float32), labels, num_segments=NUM_REGIONS)
    denom = jnp.maximum(counts, 1.0)[:, None]
    qx = jax.ops.segment_sum(h, labels, num_segments=NUM_REGIONS) / denom
    qpos = jax.ops.segment_sum(pos, labels, num_segments=NUM_REGIONS) / denom
    dmat = jnp.sum((qpos[:, None, :] - qpos[None, :, :]) ** 2, axis=-1)
    order = jnp.argsort(dmat, axis=1)
    nbrs = order[:, 1:3]
    i_idx = jnp.repeat(jnp.arange(NUM_REGIONS), 2)
    j_idx = nbrs.reshape(-1)
    row0 = jnp.stack([i_idx, j_idx], axis=1).reshape(-1)
    row1 = jnp.stack([j_idx, i_idx], axis=1).reshape(-1)
    q_ei = jnp.stack([row0, row1])
    qx = _edge_conv_small(qx, q_ei, c3_W1, c3_b1, c3_W2, c3_b2)
    qx = _edge_conv_small(qx, q_ei, c4_W1, c4_b1, c4_W2, c4_b2)
    graph_emb = jnp.sum(qx, axis=0, keepdims=True)
    out = graph_emb @ lin_W + lin_b
    return out


# speedup vs baseline: 1.1367x; 1.1367x over previous
import jax
import jax.numpy as jnp
from jax.experimental import pallas as pl

N = 10000
E = 320000
HIDDEN = 64
NUM_REGIONS = 8


def _node_transform(h, W, b):
    Np, F = h.shape
    H = W.shape[1]

    def body(h_ref, w_ref, b_ref, d_ref, bm_ref):
        hh = h_ref[...]
        wtop = w_ref[:F, :]
        wbot = w_ref[F:, :]
        bm = jnp.dot(hh, wbot, preferred_element_type=jnp.float32)
        d_ref[...] = (
            jnp.dot(hh, wtop, preferred_element_type=jnp.float32)
            - bm
            + b_ref[...]
        )
        bm_ref[...] = bm

    return pl.pallas_call(
        body,
        out_shape=(
            jax.ShapeDtypeStruct((Np, H), jnp.float32),
            jax.ShapeDtypeStruct((Np, H), jnp.float32),
        ),
    )(h, W, b.reshape(1, H))


def _edge_mlp(G1, G2, W2, b2):
    Ee, H = G1.shape
    BLK = 3200
    grid = Ee // BLK

    def body(g1_ref, g2_ref, w_ref, b_ref, o_ref):
        g = jnp.maximum(g1_ref[...] + g2_ref[...], 0.0)
        o_ref[...] = (
            jnp.dot(g, w_ref[...], preferred_element_type=jnp.float32)
            + b_ref[...]
        )

    return pl.pallas_call(
        body,
        grid=(grid,),
        in_specs=[
            pl.BlockSpec((BLK, H), lambda i: (i, 0)),
            pl.BlockSpec((BLK, H), lambda i: (i, 0)),
            pl.BlockSpec((H, H), lambda i: (0, 0)),
            pl.BlockSpec((1, H), lambda i: (0, 0)),
        ],
        out_specs=pl.BlockSpec((BLK, H), lambda i: (i, 0)),
        out_shape=jax.ShapeDtypeStruct((Ee, H), jnp.float32),
    )(G1, G2, W2, b2.reshape(1, H))


def _edge_conv_big(h, src, dst, W1, b1, W2, b2):
    D, B = _node_transform(h, W1, b1)
    G1 = jnp.take(D, dst, axis=0)
    G2 = jnp.take(B, src, axis=0)
    R = _edge_mlp(G1, G2, W2, b2)
    return jnp.maximum(jax.ops.segment_max(R, dst, num_segments=h.shape[0]), 0.0)


def _edge_conv_small(x, edge_index, W1, b1, W2, b2):
    src = edge_index[0]
    dst = edge_index[1]
    k = x.shape[0]
    oh_dst = (dst[:, None] == jnp.arange(k)[None, :]).astype(jnp.float32)
    oh_src = (src[:, None] == jnp.arange(k)[None, :]).astype(jnp.float32)
    x_i = oh_dst @ x
    x_j = oh_src @ x
    m = jnp.concatenate([x_i, x_j - x_i], axis=1)
    m = jax.nn.relu(m @ W1 + b1) @ W2 + b2
    neg = jnp.float32(-jnp.inf)
    masked = jnp.where(oh_dst[:, :, None] > 0.5, m[:, None, :], neg)
    out = jnp.max(masked, axis=0)
    return jnp.maximum(out, 0.0)


def _kmeans_onehot(pos, k, iters=10):
    cent = pos[:k]
    for _ in range(iters):
        d = jnp.sum((pos[:, None, :] - cent[None, :, :]) ** 2, axis=-1)
        labels = jnp.argmin(d, axis=1)
        oh = (labels[:, None] == jnp.arange(k)[None, :]).astype(jnp.float32)
        sums = oh.T @ pos
        counts = jnp.sum(oh, axis=0)
        cent = sums / jnp.maximum(counts, 1.0)[:, None]
    d = jnp.sum((pos[:, None, :] - cent[None, :, :]) ** 2, axis=-1)
    labels = jnp.argmin(d, axis=1)
    return (labels[:, None] == jnp.arange(k)[None, :]).astype(jnp.float32)


def kernel(x, pos, edge_index, c1_W1, c1_b1, c1_W2, c1_b2, c2_W1, c2_b1, c2_W2, c2_b2, c3_W1, c3_b1, c3_W2, c3_b2, c4_W1, c4_b1, c4_W2, c4_b2, lin_W, lin_b):
    src = edge_index[0]
    dst = edge_index[1]
    h = jnp.concatenate([x, pos], axis=1)
    h = _edge_conv_big(h, src, dst, c1_W1, c1_b1, c1_W2, c1_b2)
    h = _edge_conv_big(h, src, dst, c2_W1, c2_b1, c2_W2, c2_b2)

    oh = _kmeans_onehot(pos, NUM_REGIONS)
    counts = jnp.sum(oh, axis=0)
    denom = jnp.maximum(counts, 1.0)[:, None]
    qx = (oh.T @ h) / denom
    qpos = (oh.T @ pos) / denom
    dmat = jnp.sum((qpos[:, None, :] - qpos[None, :, :]) ** 2, axis=-1)
    order = jnp.argsort(dmat, axis=1)
    nbrs = order[:, 1:3]
    i_idx = jnp.repeat(jnp.arange(NUM_REGIONS), 2)
    j_idx = nbrs.reshape(-1)
    row0 = jnp.stack([i_idx, j_idx], axis=1).reshape(-1)
    row1 = jnp.stack([j_idx, i_idx], axis=1).reshape(-1)
    q_ei = jnp.stack([row0, row1])
    qx = _edge_conv_small(qx, q_ei, c3_W1, c3_b1, c3_W2, c3_b2)
    qx = _edge_conv_small(qx, q_ei, c4_W1, c4_b1, c4_W2, c4_b2)
    graph_emb = jnp.sum(qx, axis=0, keepdims=True)
    out = graph_emb @ lin_W + lin_b
    return out


# speedup vs baseline: 2.2379x; 1.9687x over previous
import functools

import jax
import jax.numpy as jnp
from jax import lax
from jax.experimental import pallas as pl
from jax.experimental.pallas import tpu as pltpu
from jax.experimental.pallas import tpu_sc as plsc

N = 10000
E = 320000
HIDDEN = 64
NUM_REGIONS = 8

_NC = 2
_NS = 16
_NW = _NC * _NS
_EPW = E // _NW
_GRP = 80
_BURST = 5
_CHUNK = _GRP * _BURST
_NBURSTS = _EPW // _CHUNK


def _sc_gather_pairs(T, dst, src2):
    H = T.shape[1]

    def body(t_hbm, dst_hbm, src2_hbm, g1_hbm, g2_hbm, idx1_v, idx2_v,
             rows1_v, rows2_v, gsem):
        wid = lax.axis_index("s") * _NC + lax.axis_index("c")
        base = wid * _EPW
        pltpu.sync_copy(dst_hbm.at[pl.ds(base, _EPW)], idx1_v)
        pltpu.sync_copy(src2_hbm.at[pl.ds(base, _EPW)], idx2_v)

        def step(t, carry):
            off = t * _CHUNK
            cps = []
            for u in range(_BURST):
                o = off + u * _GRP
                d = u * _GRP
                cps.append(pltpu.async_copy(
                    t_hbm.at[idx1_v.at[pl.ds(o, _GRP)]],
                    rows1_v.at[pl.ds(d, _GRP)], gsem))
                cps.append(pltpu.async_copy(
                    t_hbm.at[idx2_v.at[pl.ds(o, _GRP)]],
                    rows2_v.at[pl.ds(d, _GRP)], gsem))
            for c in cps:
                c.wait()
            pltpu.sync_copy(rows1_v, g1_hbm.at[pl.ds(base + off, _CHUNK)])
            pltpu.sync_copy(rows2_v, g2_hbm.at[pl.ds(base + off, _CHUNK)])
            return carry

        lax.fori_loop(0, _NBURSTS, step, 0)

    mesh = plsc.VectorSubcoreMesh(core_axis_name="c", subcore_axis_name="s")
    return pl.kernel(
        body,
        compiler_params=pltpu.CompilerParams(use_tc_tiling_on_sc=False),
        out_type=[
            jax.ShapeDtypeStruct((E, H), jnp.float32),
            jax.ShapeDtypeStruct((E, H), jnp.float32),
        ],
        mesh=mesh,
        scratch_types=[
            pltpu.VMEM((_EPW,), jnp.int32),
            pltpu.VMEM((_EPW,), jnp.int32),
            pltpu.VMEM((_CHUNK, H), jnp.float32),
            pltpu.VMEM((_CHUNK, H), jnp.float32),
            pltpu.SemaphoreType.DMA,
        ],
    )(T, dst, src2)


def _node_transform(h, W, b):
    Np, F = h.shape
    H = W.shape[1]

    def body(h_ref, w_ref, b_ref, t_ref):
        hh = h_ref[...]
        wtop = w_ref[:F, :]
        wbot = w_ref[F:, :]
        bm = jnp.dot(hh, wbot, preferred_element_type=jnp.float32)
        t_ref[:Np, :] = (
            jnp.dot(hh, wtop, preferred_element_type=jnp.float32)
            - bm
            + b_ref[...]
        )
        t_ref[Np:, :] = bm

    return pl.pallas_call(
        body,
        out_shape=jax.ShapeDtypeStruct((2 * Np, H), jnp.float32),
    )(h, W, b.reshape(1, H))


def _edge_mlp(G1, G2, W2, b2):
    Ee, H = G1.shape
    BLK = 3200
    grid = Ee // BLK

    def body(g1_ref, g2_ref, w_ref, b_ref, o_ref):
        g = jnp.maximum(g1_ref[...] + g2_ref[...], 0.0)
        o_ref[...] = (
            jnp.dot(g, w_ref[...], preferred_element_type=jnp.float32)
            + b_ref[...]
        )

    return pl.pallas_call(
        body,
        grid=(grid,),
        in_specs=[
            pl.BlockSpec((BLK, H), lambda i: (i, 0)),
            pl.BlockSpec((BLK, H), lambda i: (i, 0)),
            pl.BlockSpec((H, H), lambda i: (0, 0)),
            pl.BlockSpec((1, H), lambda i: (0, 0)),
        ],
        out_specs=pl.BlockSpec((BLK, H), lambda i: (i, 0)),
        out_shape=jax.ShapeDtypeStruct((Ee, H), jnp.float32),
    )(G1, G2, W2, b2.reshape(1, H))


def _edge_conv_big(h, dst, src2, W1, b1, W2, b2):
    T = _node_transform(h, W1, b1)
    G1, G2 = _sc_gather_pairs(T, dst, src2)
    R = _edge_mlp(G1, G2, W2, b2)
    return jnp.maximum(jax.ops.segment_max(R, dst, num_segments=h.shape[0]), 0.0)


def _edge_conv_small(x, edge_index, W1, b1, W2, b2):
    src = edge_index[0]
    dst = edge_index[1]
    k = x.shape[0]
    oh_dst = (dst[:, None] == jnp.arange(k)[None, :]).astype(jnp.float32)
    oh_src = (src[:, None] == jnp.arange(k)[None, :]).astype(jnp.float32)
    x_i = oh_dst @ x
    x_j = oh_src @ x
    m = jnp.concatenate([x_i, x_j - x_i], axis=1)
    m = jax.nn.relu(m @ W1 + b1) @ W2 + b2
    neg = jnp.float32(-jnp.inf)
    masked = jnp.where(oh_dst[:, :, None] > 0.5, m[:, None, :], neg)
    out = jnp.max(masked, axis=0)
    return jnp.maximum(out, 0.0)


def _kmeans_onehot(pos, k, iters=10):
    cent = pos[:k]
    for _ in range(iters):
        d = jnp.sum((pos[:, None, :] - cent[None, :, :]) ** 2, axis=-1)
        labels = jnp.argmin(d, axis=1)
        oh = (labels[:, None] == jnp.arange(k)[None, :]).astype(jnp.float32)
        sums = oh.T @ pos
        counts = jnp.sum(oh, axis=0)
        cent = sums / jnp.maximum(counts, 1.0)[:, None]
    d = jnp.sum((pos[:, None, :] - cent[None, :, :]) ** 2, axis=-1)
    labels = jnp.argmin(d, axis=1)
    return (labels[:, None] == jnp.arange(k)[None, :]).astype(jnp.float32)


def kernel(x, pos, edge_index, c1_W1, c1_b1, c1_W2, c1_b2, c2_W1, c2_b1, c2_W2, c2_b2, c3_W1, c3_b1, c3_W2, c3_b2, c4_W1, c4_b1, c4_W2, c4_b2, lin_W, lin_b):
    src = edge_index[0]
    dst = edge_index[1]
    src2 = src + N
    h = jnp.concatenate([x, pos], axis=1)
    h = _edge_conv_big(h, dst, src2, c1_W1, c1_b1, c1_W2, c1_b2)
    h = _edge_conv_big(h, dst, src2, c2_W1, c2_b1, c2_W2, c2_b2)

    oh = _kmeans_onehot(pos, NUM_REGIONS)
    counts = jnp.sum(oh, axis=0)
    denom = jnp.maximum(counts, 1.0)[:, None]
    qx = (oh.T @ h) / denom
    qpos = (oh.T @ pos) / denom
    dmat = jnp.sum((qpos[:, None, :] - qpos[None, :, :]) ** 2, axis=-1)
    order = jnp.argsort(dmat, axis=1)
    nbrs = order[:, 1:3]
    i_idx = jnp.repeat(jnp.arange(NUM_REGIONS), 2)
    j_idx = nbrs.reshape(-1)
    row0 = jnp.stack([i_idx, j_idx], axis=1).reshape(-1)
    row1 = jnp.stack([j_idx, i_idx], axis=1).reshape(-1)
    q_ei = jnp.stack([row0, row1])
    qx = _edge_conv_small(qx, q_ei, c3_W1, c3_b1, c3_W2, c3_b2)
    qx = _edge_conv_small(qx, q_ei, c4_W1, c4_b1, c4_W2, c4_b2)
    graph_emb = jnp.sum(qx, axis=0, keepdims=True)
    out = graph_emb @ lin_W + lin_b
    return out
